# baseline (speedup 1.0000x reference)
import functools
import math

import jax
import jax.numpy as jnp
from jax.experimental import pallas as pl

TI = 8
TJ = 1024


def _feature_body(x_ref, g_ref, b_ref, wp_ref, bp_ref, wq_ref, wk_ref,
                  q_ref, k_ref):
    x = x_ref[...].astype(jnp.float32)
    mu = jnp.mean(x, axis=1, keepdims=True)
    d = x - mu
    var = jnp.mean(d * d, axis=1, keepdims=True)
    xn = d * jax.lax.rsqrt(var + 1e-5) * g_ref[...] + b_ref[...]
    f = jnp.dot(xn.astype(jnp.bfloat16), wp_ref[...],
                preferred_element_type=jnp.float32) + bp_ref[...]
    f = jnp.maximum(f, 0.0).astype(jnp.bfloat16)
    q = jnp.dot(f, wq_ref[...], preferred_element_type=jnp.float32)
    k = jnp.dot(f, wk_ref[...], preferred_element_type=jnp.float32)
    q_ref[...] = q.astype(jnp.bfloat16)
    k_ref[...] = k.astype(jnp.bfloat16)


def _pair_body(scale, bxi_ref, bxt_ref, q_ref, k_ref, w1t_ref, sgn_ref,
               b2_ref, roff_ref, out_ref):
    pj = pl.program_id(0)
    pi = pl.program_id(1)

    x1j = bxt_ref[0:1, :]
    y1j = bxt_ref[1:2, :]
    x2j = bxt_ref[2:3, :]
    y2j = bxt_ref[3:4, :]
    wj = jnp.maximum(x2j - x1j, 1.0)
    hj = jnp.maximum(y2j - y1j, 1.0)
    cxj = (x1j + x2j) * 0.5
    cyj = (y1j + y2j) * 0.5
    iwj = 1.0 / wj
    ihj = 1.0 / hj

    qk = jax.lax.dot_general(
        q_ref[...], k_ref[...], (((1,), (1,)), ((), ())),
        preferred_element_type=jnp.float32) * scale

    w1t = w1t_ref[...]
    sgn = sgn_ref[...]
    ones = jnp.ones_like(x1j)

    x1i = bxi_ref[:, 0:1]
    y1i = bxi_ref[:, 1:2]
    x2i = bxi_ref[:, 2:3]
    y2i = bxi_ref[:, 3:4]
    wi = jnp.maximum(x2i - x1i, 1.0)
    hi = jnp.maximum(y2i - y1i, 1.0)
    cxi = (x1i + x2i) * 0.5
    cyi = (y1i + y2i) * 0.5
    iwi = 1.0 / wi
    ihi = 1.0 / hi

    m1 = jnp.minimum(x1i, x1j)
    m2 = jnp.minimum(y1i, y1j)
    M1 = jnp.maximum(x2i, x2j)
    M2 = jnp.maximum(y2i, y2j)
    wu = jnp.maximum(M1 - m1, 1.0)
    hu = jnp.maximum(M2 - m2, 1.0)
    cxu = (m1 + M1) * 0.5
    cyu = (m2 + M2) * 0.5

    cs = [
        (cxj - cxi) * iwi,
        (cyj - cyi) * ihi,
        (wj - wi) * iwi,
        (hj - hi) * ihi,
        jnp.log(wj * iwi + 1e-6),
        jnp.log(hj * ihi + 1e-6),
        (cxu - cxi) * iwi,
        (cyu - cyi) * ihi,
        (wu - wi) * iwi,
        (hu - hi) * ihi,
        jnp.log(wu * iwi + 1e-6),
        jnp.log(hu * ihi + 1e-6),
        (cxu - cxj) * iwj,
        (cyu - cyj) * ihj,
        (wu - wj) * iwj,
        (hu - hj) * ihj,
        jnp.log(wu * iwj + 1e-6),
        jnp.log(hu * ihj + 1e-6),
    ]

    rows = []
    for r in range(TI):
        g = jnp.concatenate(
            [c[r:r + 1, :] for c in cs] + [ones],
            axis=0).astype(jnp.float8_e4m3fn)
        z = jnp.dot(w1t, g, preferred_element_type=jnp.float32)
        a = jnp.maximum(z.astype(jnp.bfloat16), jnp.bfloat16(0.0)) * sgn
        r = a.shape[0]
        while r > 16:
            r //= 2
            a = a[:r] + a[r:]
        rows.append(jnp.sum(a.astype(jnp.float32), axis=0,
                            keepdims=True))

    spatial = jnp.concatenate(rows, axis=0)

    ri = roff_ref[0, 0] + pi * TI + jax.lax.broadcasted_iota(
        jnp.int32, (TI, TJ), 0)
    cj = pj * TJ + jax.lax.broadcasted_iota(jnp.int32, (TI, TJ), 1)
    diag = jnp.where(ri == cj, -1e9, 0.0)
    out_ref[...] = qk + spatial + b2_ref[0, 0] + diag


def _feature_call(feats_loc, g2, bt2, wcat, bp, rows_local, H):
    q_loc, k_loc = pl.pallas_call(
        _feature_body,
        out_shape=[jax.ShapeDtypeStruct((rows_local, H), jnp.bfloat16),
                   jax.ShapeDtypeStruct((rows_local, H), jnp.bfloat16)],
    )(feats_loc, g2, bt2, wcat[:, 0:H], bp, wcat[:, H:2 * H],
      wcat[:, 2 * H:3 * H])
    return q_loc, k_loc


def _pair_call(bx_loc, bxt, q_loc, k_bf, w1t_aug, sgn, b2c, roff,
               rows_local, H, MLP_H, Np):
    roff_arr = jnp.reshape(roff, (1, 1)).astype(jnp.int32)
    grid = (Np // TJ, rows_local // TI)
    return pl.pallas_call(
        functools.partial(_pair_body, float(1.0 / math.sqrt(H))),
        grid=grid,
        in_specs=[
            pl.BlockSpec((TI, 4), lambda pj, pi: (pi, 0)),
            pl.BlockSpec((4, TJ), lambda pj, pi: (0, pj)),
            pl.BlockSpec((TI, H), lambda pj, pi: (pi, 0)),
            pl.BlockSpec((TJ, H), lambda pj, pi: (pj, 0)),
            pl.BlockSpec((MLP_H, 19), lambda pj, pi: (0, 0)),
            pl.BlockSpec((MLP_H, 1), lambda pj, pi: (0, 0)),
            pl.BlockSpec((1, 1), lambda pj, pi: (0, 0)),
            pl.BlockSpec((1, 1), lambda pj, pi: (0, 0)),
        ],
        out_specs=pl.BlockSpec((TI, TJ), lambda pj, pi: (pi, pj)),
        out_shape=jax.ShapeDtypeStruct((rows_local, Np), jnp.float32),
    )(bx_loc, bxt, q_loc, k_bf, w1t_aug, sgn, b2c, roff_arr)


def kernel(features, boxes, ln_gamma, ln_beta, W_proj, b_proj, W_q, W_k,
           W1, b1, W2, b2):
    B, N, H = features.shape
    MLP_H = W1.shape[1]
    Np = ((N + TJ - 1) // TJ) * TJ

    feats = jnp.pad(features[0], ((0, Np - N), (0, 0))).astype(jnp.bfloat16)
    padbox = jnp.tile(jnp.array([[0.0, 0.0, 16.0, 16.0]], jnp.float32),
                      (Np - N, 1))
    bx = jnp.concatenate([boxes[0], padbox], axis=0)
    bxt = bx.T

    w2v = W2[:, 0]
    w2a = jnp.abs(w2v)
    w1t_aug = (jnp.concatenate(
        [W1.T * w2a[:, None], (b1 * w2a)[:, None]], axis=1)
        * 64.0).astype(jnp.float8_e4m3fn)
    sgn = (jnp.where(w2v >= 0, 1.0, -1.0) / 64.0)[:, None].astype(jnp.bfloat16)

    wcat = jnp.concatenate([W_proj, W_q, W_k], axis=1).astype(jnp.bfloat16)
    args = (feats, bx, bxt, ln_gamma[None, :], ln_beta[None, :], wcat,
            b_proj[None, :], w1t_aug, sgn, b2[None, :])

    nd = 2 if (jax.device_count() >= 2 and Np % (2 * TI) == 0) else 1
    if nd == 1:
        q_loc, k_loc = _feature_call(feats, ln_gamma[None, :],
                                     ln_beta[None, :], wcat, b_proj[None, :],
                                     Np, H)
        out = _pair_call(bx, bxt, q_loc, k_loc, w1t_aug, sgn, b2[None, :],
                         jnp.int32(0), Np, H, MLP_H, Np)
    else:
        from jax.sharding import Mesh, PartitionSpec as P
        import numpy as np
        mesh = Mesh(np.array(jax.devices()[:nd]), ("x",))
        rows_local = Np // nd

        def shard_fn(feats_l, bx_l, bxt_f, g2, bt2, wcat_l, bp, w1t, sg, b2c):
            roff = (jax.lax.axis_index("x") * rows_local).astype(jnp.int32)
            wcat_f = jax.lax.all_gather(wcat_l, "x", axis=0, tiled=True)
            q_l, k_l = _feature_call(feats_l, g2, bt2, wcat_f, bp,
                                     rows_local, H)
            k_f = jax.lax.all_gather(k_l, "x", axis=0, tiled=True)
            return _pair_call(bx_l, bxt_f, q_l, k_f, w1t, sg, b2c, roff,
                              rows_local, H, MLP_H, Np)

        out = jax.shard_map(
            shard_fn, mesh=mesh,
            in_specs=(P("x", None), P("x", None), P(), P(), P(),
                      P("x", None), P(), P(), P(), P()),
            out_specs=P("x", None),
            check_vma=False,
        )(*args)

    return out[None, :N, :N]


# speedup vs baseline: 1.9813x; 1.9813x over previous
import functools
import math

import jax
import jax.numpy as jnp
from jax.experimental import pallas as pl

TI = 8
TJ = 1024


def _feature_body(x_ref, g_ref, b_ref, wp_ref, bp_ref, wq_ref, wk_ref,
                  q_ref, k_ref):
    x = x_ref[...].astype(jnp.float32)
    mu = jnp.mean(x, axis=1, keepdims=True)
    d = x - mu
    var = jnp.mean(d * d, axis=1, keepdims=True)
    xn = d * jax.lax.rsqrt(var + 1e-5) * g_ref[...] + b_ref[...]
    f = jnp.dot(xn.astype(jnp.bfloat16), wp_ref[...],
                preferred_element_type=jnp.float32) + bp_ref[...]
    f = jnp.maximum(f, 0.0).astype(jnp.bfloat16)
    q = jnp.dot(f, wq_ref[...], preferred_element_type=jnp.float32)
    k = jnp.dot(f, wk_ref[...], preferred_element_type=jnp.float32)
    q_ref[...] = q.astype(jnp.bfloat16)
    k_ref[...] = k.astype(jnp.bfloat16)


def _pair_body(scale, bxi_ref, bxt_ref, q_ref, k_ref, w1t_ref, sgn_ref,
               b2_ref, roff_ref, out_ref):
    pj = pl.program_id(0)
    pi = pl.program_id(1)

    x1j = bxt_ref[0:1, :]
    y1j = bxt_ref[1:2, :]
    x2j = bxt_ref[2:3, :]
    y2j = bxt_ref[3:4, :]
    wj = jnp.maximum(x2j - x1j, 1.0)
    hj = jnp.maximum(y2j - y1j, 1.0)
    cxj = (x1j + x2j) * 0.5
    cyj = (y1j + y2j) * 0.5
    iwj = 1.0 / wj
    ihj = 1.0 / hj

    qk = jax.lax.dot_general(
        q_ref[...], k_ref[...], (((1,), (1,)), ((), ())),
        preferred_element_type=jnp.float32) * scale

    w1t = w1t_ref[...]
    sgn = sgn_ref[...]
    ones = jnp.ones_like(x1j)

    x1i = bxi_ref[:, 0:1]
    y1i = bxi_ref[:, 1:2]
    x2i = bxi_ref[:, 2:3]
    y2i = bxi_ref[:, 3:4]
    wi = jnp.maximum(x2i - x1i, 1.0)
    hi = jnp.maximum(y2i - y1i, 1.0)
    cxi = (x1i + x2i) * 0.5
    cyi = (y1i + y2i) * 0.5
    iwi = 1.0 / wi
    ihi = 1.0 / hi

    m1 = jnp.minimum(x1i, x1j)
    m2 = jnp.minimum(y1i, y1j)
    M1 = jnp.maximum(x2i, x2j)
    M2 = jnp.maximum(y2i, y2j)
    wu = jnp.maximum(M1 - m1, 1.0)
    hu = jnp.maximum(M2 - m2, 1.0)
    cxu = (m1 + M1) * 0.5
    cyu = (m2 + M2) * 0.5

    cs = [
        (cxj - cxi) * iwi,
        (cyj - cyi) * ihi,
        (wj - wi) * iwi,
        (hj - hi) * ihi,
        jnp.log(wj * iwi + 1e-6),
        jnp.log(hj * ihi + 1e-6),
        (cxu - cxi) * iwi,
        (cyu - cyi) * ihi,
        (wu - wi) * iwi,
        (hu - hi) * ihi,
        jnp.log(wu * iwi + 1e-6),
        jnp.log(hu * ihi + 1e-6),
        (cxu - cxj) * iwj,
        (cyu - cyj) * ihj,
        (wu - wj) * iwj,
        (hu - hj) * ihj,
        jnp.log(wu * iwj + 1e-6),
        jnp.log(hu * ihj + 1e-6),
    ]

    rows = []
    for r in range(TI):
        g = jnp.concatenate(
            [c[r:r + 1, :] for c in cs] + [ones],
            axis=0).astype(jnp.float8_e4m3fn)
        z = jnp.dot(w1t, g, preferred_element_type=jnp.float32)
        a = jnp.maximum(z.astype(jnp.bfloat16), jnp.bfloat16(0.0)) * sgn
        r = a.shape[0]
        while r > 16:
            r //= 2
            a = a[:r] + a[r:]
        rows.append(jnp.sum(a.astype(jnp.float32), axis=0,
                            keepdims=True))

    spatial = jnp.concatenate(rows, axis=0)

    ri = roff_ref[0, 0] + pi * TI + jax.lax.broadcasted_iota(
        jnp.int32, (TI, TJ), 0)
    cj = pj * TJ + jax.lax.broadcasted_iota(jnp.int32, (TI, TJ), 1)
    diag = jnp.where(ri == cj, -1e9, 0.0)
    out_ref[...] = qk + spatial + b2_ref[0, 0] + diag


def _feature_call(feats_loc, g2, bt2, wcat, bp, rows_local, H):
    q_loc, k_loc = pl.pallas_call(
        _feature_body,
        out_shape=[jax.ShapeDtypeStruct((rows_local, H), jnp.bfloat16),
                   jax.ShapeDtypeStruct((rows_local, H), jnp.bfloat16)],
    )(feats_loc, g2, bt2, wcat[:, 0:H], bp, wcat[:, H:2 * H],
      wcat[:, 2 * H:3 * H])
    return q_loc, k_loc


def _pair_call(bx_loc, bxt, q_loc, k_bf, w1t_aug, sgn, b2c, roff,
               rows_local, H, MLP_H, Np):
    roff_arr = jnp.reshape(roff, (1, 1)).astype(jnp.int32)
    grid = (Np // TJ, rows_local // TI)
    return pl.pallas_call(
        functools.partial(_pair_body, float(1.0 / math.sqrt(H))),
        grid=grid,
        in_specs=[
            pl.BlockSpec((TI, 4), lambda pj, pi: (pi, 0)),
            pl.BlockSpec((4, TJ), lambda pj, pi: (0, pj)),
            pl.BlockSpec((TI, H), lambda pj, pi: (pi, 0)),
            pl.BlockSpec((TJ, H), lambda pj, pi: (pj, 0)),
            pl.BlockSpec((MLP_H, 19), lambda pj, pi: (0, 0)),
            pl.BlockSpec((MLP_H, 1), lambda pj, pi: (0, 0)),
            pl.BlockSpec((1, 1), lambda pj, pi: (0, 0)),
            pl.BlockSpec((1, 1), lambda pj, pi: (0, 0)),
        ],
        out_specs=pl.BlockSpec((TI, TJ), lambda pj, pi: (pi, pj)),
        out_shape=jax.ShapeDtypeStruct((rows_local, Np), jnp.float32),
    )(bx_loc, bxt, q_loc, k_bf, w1t_aug, sgn, b2c, roff_arr)


def kernel(features, boxes, ln_gamma, ln_beta, W_proj, b_proj, W_q, W_k,
           W1, b1, W2, b2):
    B, N, H = features.shape
    MLP_H = W1.shape[1]
    Np = ((N + TJ - 1) // TJ) * TJ

    feats = jnp.pad(features[0], ((0, Np - N), (0, 0))).astype(jnp.bfloat16)
    padbox = jnp.tile(jnp.array([[0.0, 0.0, 16.0, 16.0]], jnp.float32),
                      (Np - N, 1))
    bx = jnp.concatenate([boxes[0], padbox], axis=0)
    bxt = bx.T

    w2v = W2[:, 0]
    w2a = jnp.abs(w2v)
    w1t_aug = (jnp.concatenate(
        [W1.T * w2a[:, None], (b1 * w2a)[:, None]], axis=1)
        * 64.0).astype(jnp.float8_e4m3fn)
    sgn = (jnp.where(w2v >= 0, 1.0, -1.0) / 64.0)[:, None].astype(jnp.bfloat16)

    wcat = jnp.concatenate([W_proj, W_q, W_k], axis=1).astype(jnp.bfloat16)
    args = (feats, bx, bxt, ln_gamma[None, :], ln_beta[None, :], wcat,
            b_proj[None, :], w1t_aug, sgn, b2[None, :])

    nd = 2 if (jax.device_count() >= 2 and Np % (2 * TI) == 0) else 1
    if nd == 1:
        q_loc, k_loc = _feature_call(feats, ln_gamma[None, :],
                                     ln_beta[None, :], wcat, b_proj[None, :],
                                     Np, H)
        out = _pair_call(bx, bxt, q_loc, k_loc, w1t_aug, sgn, b2[None, :],
                         jnp.int32(0), Np, H, MLP_H, Np)
    else:
        from jax.sharding import Mesh, PartitionSpec as P
        import numpy as np
        mesh = Mesh(np.array(jax.devices()[:nd]), ("x",))
        rows_local = Np // nd

        def shard_fn(feats_f, bx_f, bxt_f, g2, bt2, wcat_f, bp, w1t, sg, b2c):
            roff = (jax.lax.axis_index("x") * rows_local).astype(jnp.int32)
            q_f, k_f = _feature_call(feats_f, g2, bt2, wcat_f, bp, Np, H)
            q_l = jax.lax.dynamic_slice_in_dim(q_f, roff, rows_local, 0)
            bx_l = jax.lax.dynamic_slice_in_dim(bx_f, roff, rows_local, 0)
            return _pair_call(bx_l, bxt_f, q_l, k_f, w1t, sg, b2c, roff,
                              rows_local, H, MLP_H, Np)

        out = jax.shard_map(
            shard_fn, mesh=mesh,
            in_specs=(P(),) * len(args),
            out_specs=P("x", None),
            check_vma=False,
        )(*args)

    return out[None, :N, :N]
